# baseline (speedup 1.0000x reference)
import functools

import jax
import jax.numpy as jnp
from jax.experimental import pallas as pl
from jax.experimental.pallas import tpu as pltpu

_SCALE = 10.0
_MARGIN = 0.35


def _amsoftmax_kernel(x_ref, tgt_ref, out_ref, m_ref, s_ref, t_ref, *,
                      num_classes, num_rows):
    r = pl.program_id(0)
    c = pl.program_id(1)
    nc = pl.num_programs(1)
    w = x_ref.shape[1]

    x = x_ref[...] * _SCALE
    gcol = c * w + jax.lax.broadcasted_iota(jnp.int32, x.shape, 1)
    tgt = tgt_ref[...]
    is_t = gcol == tgt
    xs = x - (_MARGIN * _SCALE) * is_t.astype(x.dtype)
    xs = jnp.where(gcol < num_classes, xs, -jnp.inf)

    bmax = jnp.max(xs, axis=1, keepdims=True)
    bsum = jnp.sum(jnp.exp(xs - bmax), axis=1, keepdims=True)
    tval = jnp.sum(jnp.where(is_t, xs, 0.0), axis=1, keepdims=True)

    @pl.when(c == 0)
    def _init():
        m_ref[...] = jnp.full_like(m_ref, -jnp.inf)
        s_ref[...] = jnp.zeros_like(s_ref)
        t_ref[...] = jnp.zeros_like(t_ref)

    m_old = m_ref[...]
    m_new = jnp.maximum(m_old, bmax)
    s_ref[...] = s_ref[...] * jnp.exp(m_old - m_new) + bsum * jnp.exp(bmax - m_new)
    m_ref[...] = m_new
    t_ref[...] = t_ref[...] + tval

    @pl.when(c == nc - 1)
    def _finish():
        row_loss = (m_ref[...] + jnp.log(s_ref[...])) - t_ref[...]
        part = jnp.sum(row_loss, keepdims=True).reshape(1, 1) * (1.0 / num_rows)

        @pl.when(r == 0)
        def _():
            out_ref[...] = part

        @pl.when(r != 0)
        def _():
            out_ref[...] = out_ref[...] + part


def kernel(input, target):
    n_rows, n_cls = input.shape
    block_r = min(256, n_rows)
    block_w = min(2048, n_cls)
    grid = (pl.cdiv(n_rows, block_r), pl.cdiv(n_cls, block_w))

    tgt2d = target.astype(jnp.int32).reshape(n_rows, 1)

    out = pl.pallas_call(
        functools.partial(_amsoftmax_kernel, num_classes=n_cls,
                          num_rows=n_rows),
        grid=grid,
        in_specs=[
            pl.BlockSpec((block_r, block_w), lambda r, c: (r, c)),
            pl.BlockSpec((block_r, 1), lambda r, c: (r, 0)),
        ],
        out_specs=pl.BlockSpec((1, 1), lambda r, c: (0, 0)),
        out_shape=jax.ShapeDtypeStruct((1, 1), jnp.float32),
        scratch_shapes=[
            pltpu.VMEM((block_r, 1), jnp.float32),
            pltpu.VMEM((block_r, 1), jnp.float32),
            pltpu.VMEM((block_r, 1), jnp.float32),
        ],
    )(input, tgt2d)
    return out[0, 0]


# speedup vs baseline: 1.1621x; 1.1621x over previous
import functools
import math

import jax
import jax.numpy as jnp
from jax.experimental import pallas as pl
from jax.experimental.pallas import tpu as pltpu

_SCALE = 10.0
_MARGIN = 0.35
_SM = _SCALE * _MARGIN
_EM1 = math.expm1(-_SM)


def _amsoftmax_kernel(x_ref, tgt_ref, out_ref, m_ref, s_ref, t_ref, *,
                      num_rows, num_classes):
    r = pl.program_id(0)
    c = pl.program_id(1)
    nc = pl.num_programs(1)
    w = x_ref.shape[1]

    @pl.when(c == 0)
    def _init():
        m_ref[...] = jnp.full_like(m_ref, -jnp.inf)
        s_ref[...] = jnp.zeros_like(s_ref)
        t_ref[...] = jnp.zeros_like(t_ref)

    def _update(masked):
        x = x_ref[...]
        tgt = tgt_ref[...]
        lcol = jax.lax.broadcasted_iota(jnp.int32, x.shape, 1)
        is_t = lcol == (tgt - c * w)
        tval = jnp.sum(jnp.where(is_t, x, 0.0), axis=1, keepdims=True)
        if masked:
            xm = jnp.where(lcol < num_classes - c * w, x, -jnp.inf)
        else:
            xm = x
        bmax = jnp.max(xm, axis=1, keepdims=True) * _SCALE
        m_old = m_ref[...]
        m_new = jnp.maximum(m_old, bmax)
        bsum = jnp.sum(jnp.exp(xm * _SCALE - m_new), axis=1, keepdims=True)
        s_ref[...] = s_ref[...] * jnp.exp(m_old - m_new) + bsum
        m_ref[...] = m_new
        t_ref[...] = t_ref[...] + tval

    ragged = num_classes % w != 0

    @pl.when(c < nc - 1)
    def _full():
        _update(masked=False)

    @pl.when(c == nc - 1)
    def _last():
        _update(masked=ragged)

    @pl.when(c == nc - 1)
    def _finish():
        lse = m_ref[...] + jnp.log(s_ref[...])
        a_t = t_ref[...] * _SCALE
        row_loss = lse + jnp.log(1.0 + _EM1 * jnp.exp(a_t - lse)) - a_t + _SM
        part = jnp.sum(row_loss, keepdims=True).reshape(1, 1) * (1.0 / num_rows)

        @pl.when(r == 0)
        def _():
            out_ref[...] = part

        @pl.when(r != 0)
        def _():
            out_ref[...] = out_ref[...] + part


def kernel(input, target):
    n_rows, n_cls = input.shape
    block_r = min(256, n_rows)
    block_w = min(4096, n_cls)
    grid = (pl.cdiv(n_rows, block_r), pl.cdiv(n_cls, block_w))

    tgt2d = target.astype(jnp.int32).reshape(n_rows, 1)

    out = pl.pallas_call(
        functools.partial(_amsoftmax_kernel, num_rows=n_rows,
                          num_classes=n_cls),
        grid=grid,
        in_specs=[
            pl.BlockSpec((block_r, block_w), lambda r, c: (r, c)),
            pl.BlockSpec((block_r, 1), lambda r, c: (r, 0)),
        ],
        out_specs=pl.BlockSpec((1, 1), lambda r, c: (0, 0)),
        out_shape=jax.ShapeDtypeStruct((1, 1), jnp.float32),
        scratch_shapes=[
            pltpu.VMEM((block_r, 1), jnp.float32),
            pltpu.VMEM((block_r, 1), jnp.float32),
            pltpu.VMEM((block_r, 1), jnp.float32),
        ],
    )(input, tgt2d)
    return out[0, 0]
